# baseline (speedup 1.0000x reference)
import dataclasses
import functools

import jax
import jax.numpy as jnp
from jax import lax
from jax.experimental import pallas as pl
from jax.experimental.pallas import tpu as pltpu
from jax.experimental.pallas import tpu_sc as plsc

M = 2048
BATCH_BLOCK = 512
NW = 32
SC_ROWS = 4096


def _tc_body(h_ref, sign_ref, x_ref, out_ref, s_scratch):
    d = h_ref.shape[0]
    step = pl.program_id(0)

    @pl.when(step == 0)
    def _build():
        bins = jax.lax.broadcasted_iota(jnp.int32, (d, M), 1)
        onehot = jnp.where(h_ref[...] == bins, sign_ref[...], 0.0)
        s_scratch[...] = onehot.astype(jnp.bfloat16)

    @pl.when(step > 0)
    def _dot():
        out_ref[...] = jnp.dot(
            x_ref[...].astype(jnp.bfloat16),
            s_scratch[...],
            preferred_element_type=jnp.float32,
        )


def _tc_countsketch(x, h2, sign2):
    batch, d = x.shape
    nb = batch // BATCH_BLOCK
    blk = lambda b: jnp.maximum(b - 1, 0)
    return pl.pallas_call(
        _tc_body,
        grid=(nb + 1,),
        in_specs=[
            pl.BlockSpec((d, 1), lambda b: (0, 0)),
            pl.BlockSpec((d, 1), lambda b: (0, 0)),
            pl.BlockSpec((BATCH_BLOCK, d), lambda b: (blk(b), 0)),
        ],
        out_specs=pl.BlockSpec((BATCH_BLOCK, M), lambda b: (blk(b), 0)),
        out_shape=jax.ShapeDtypeStruct((batch, M), x.dtype),
        scratch_shapes=[pltpu.VMEM((d, M), jnp.bfloat16)],
        compiler_params=pltpu.CompilerParams(
            dimension_semantics=("arbitrary",),
        ),
    )(h2, sign2, x)


def _sc_body(x_hbm, h_hbm, sign_hbm, out_hbm, h_v, sign_v, x_v, acc_v):
    d = h_hbm.shape[0]
    wid = lax.axis_index("s") * 2 + lax.axis_index("c")
    rows_per_w = x_hbm.shape[0] // NW
    base = wid * rows_per_w

    pltpu.sync_copy(h_hbm, h_v)
    pltpu.sync_copy(sign_hbm, sign_v)

    @pl.loop(0, rows_per_w)
    def _row(r):
        row = base + r
        pltpu.sync_copy(x_hbm.at[row], x_v)

        @pl.loop(0, M, step=16)
        def _zero(i):
            acc_v[pl.ds(i, 16)] = jnp.zeros((16,), jnp.float32)

        @pl.loop(0, d, step=16)
        def _grp(g):
            v = x_v[pl.ds(g, 16)] * sign_v[pl.ds(g, 16)]
            idx = h_v[pl.ds(g, 16)]
            plsc.addupdate_scatter(acc_v, [idx], v)

        pltpu.sync_copy(acc_v, out_hbm.at[row])


def _sc_countsketch(x, h32, sign):
    batch, d = x.shape
    mesh = plsc.VectorSubcoreMesh(core_axis_name="c", subcore_axis_name="s")
    cp = pltpu.CompilerParams()
    if "needs_layout_passes" in pltpu.CompilerParams.__dataclass_fields__:
        cp = dataclasses.replace(cp, needs_layout_passes=False)
    f = functools.partial(
        pl.kernel,
        out_type=jax.ShapeDtypeStruct((batch, M), jnp.float32),
        mesh=mesh,
        compiler_params=cp,
        scratch_types=[
            pltpu.VMEM((d,), jnp.int32),
            pltpu.VMEM((d,), jnp.float32),
            pltpu.VMEM((d,), jnp.float32),
            pltpu.VMEM((M,), jnp.float32),
        ],
    )(_sc_body)
    return f(x, h32, sign)


def kernel(x, h, sign):
    batch, d = x.shape
    h32 = h.astype(jnp.int32)
    parts = []
    if SC_ROWS < batch:
        h2 = h32.reshape(d, 1)
        sign2 = sign.reshape(d, 1)
        parts.append(_tc_countsketch(x[: batch - SC_ROWS], h2, sign2))
    if SC_ROWS > 0:
        parts.append(_sc_countsketch(x[batch - SC_ROWS :], h32, sign))
    if len(parts) == 1:
        return parts[0]
    return jnp.concatenate(parts, axis=0)


# speedup vs baseline: 1.8027x; 1.8027x over previous
import dataclasses
import functools

import jax
import jax.numpy as jnp
from jax import lax
from jax.experimental import pallas as pl
from jax.experimental.pallas import tpu as pltpu
from jax.experimental.pallas import tpu_sc as plsc

M = 2048
BATCH_BLOCK = 512
NW = 32
SC_ROWS = 4096


def _tc_body(h_ref, sign_ref, x_ref, out_ref, s_scratch):
    d = h_ref.shape[0]
    step = pl.program_id(0)

    @pl.when(step == 0)
    def _build():
        bins = jax.lax.broadcasted_iota(jnp.int32, (d, M), 1)
        onehot = jnp.where(h_ref[...] == bins, sign_ref[...], 0.0)
        s_scratch[...] = onehot.astype(jnp.bfloat16)

    @pl.when(step > 0)
    def _dot():
        out_ref[...] = jnp.dot(
            x_ref[...].astype(jnp.bfloat16),
            s_scratch[...],
            preferred_element_type=jnp.float32,
        )


def _tc_countsketch(x, h2, sign2):
    batch, d = x.shape
    nb = batch // BATCH_BLOCK
    blk = lambda b: jnp.maximum(b - 1, 0)
    return pl.pallas_call(
        _tc_body,
        grid=(nb + 1,),
        in_specs=[
            pl.BlockSpec((d, 1), lambda b: (0, 0)),
            pl.BlockSpec((d, 1), lambda b: (0, 0)),
            pl.BlockSpec((BATCH_BLOCK, d), lambda b: (blk(b), 0)),
        ],
        out_specs=pl.BlockSpec((BATCH_BLOCK, M), lambda b: (blk(b), 0)),
        out_shape=jax.ShapeDtypeStruct((batch, M), x.dtype),
        scratch_shapes=[pltpu.VMEM((d, M), jnp.bfloat16)],
        compiler_params=pltpu.CompilerParams(
            dimension_semantics=("arbitrary",),
        ),
    )(h2, sign2, x)


ROWS_PER_CHUNK = 8


def _sc_body(x_hbm, h_hbm, sign_hbm, out_hbm, h_v, sign_v, xa_v, xb_v, acc_v,
             sem_a, sem_b):
    d = h_hbm.shape[0]
    wid = lax.axis_index("s") * 2 + lax.axis_index("c")
    rows_per_w = x_hbm.shape[0] // NW
    n_chunks = rows_per_w // ROWS_PER_CHUNK
    base = wid * rows_per_w

    pltpu.sync_copy(h_hbm, h_v)
    pltpu.sync_copy(sign_hbm, sign_v)

    bufs = (xa_v, xb_v)
    sems = (sem_a, sem_b)

    def start(t):
        return pltpu.async_copy(
            x_hbm.at[pl.ds(base + t * ROWS_PER_CHUNK, ROWS_PER_CHUNK)],
            bufs[t % 2],
            sems[t % 2],
        )

    handles = {0: start(0)}
    for t in range(n_chunks):
        if t + 1 < n_chunks:
            handles[t + 1] = start(t + 1)
        handles[t].wait()
        xbuf = bufs[t % 2]

        @pl.loop(0, M, step=16)
        def _zero(i):
            z = jnp.zeros((16,), jnp.float32)
            for r in range(ROWS_PER_CHUNK):
                acc_v[r, pl.ds(i, 16)] = z

        @pl.loop(0, d, step=16)
        def _grp(g):
            idx = h_v[pl.ds(g, 16)]
            sgn = sign_v[pl.ds(g, 16)]
            for r in range(ROWS_PER_CHUNK):
                v = xbuf[r, pl.ds(g, 16)] * sgn
                row_idx = jnp.full((16,), r, jnp.int32)
                plsc.addupdate_scatter(acc_v, [row_idx, idx], v)

        pltpu.sync_copy(
            acc_v,
            out_hbm.at[pl.ds(base + t * ROWS_PER_CHUNK, ROWS_PER_CHUNK)],
        )


def _sc_countsketch(x, h32, sign):
    batch, d = x.shape
    mesh = plsc.VectorSubcoreMesh(core_axis_name="c", subcore_axis_name="s")
    cp = pltpu.CompilerParams()
    if "needs_layout_passes" in pltpu.CompilerParams.__dataclass_fields__:
        cp = dataclasses.replace(cp, needs_layout_passes=False)
    f = functools.partial(
        pl.kernel,
        out_type=jax.ShapeDtypeStruct((batch, M), jnp.float32),
        mesh=mesh,
        compiler_params=cp,
        scratch_types=[
            pltpu.VMEM((d,), jnp.int32),
            pltpu.VMEM((d,), jnp.float32),
            pltpu.VMEM((ROWS_PER_CHUNK, d), jnp.float32),
            pltpu.VMEM((ROWS_PER_CHUNK, d), jnp.float32),
            pltpu.VMEM((ROWS_PER_CHUNK, M), jnp.float32),
            pltpu.SemaphoreType.DMA,
            pltpu.SemaphoreType.DMA,
        ],
    )(_sc_body)
    return f(x, h32, sign)


def kernel(x, h, sign):
    batch, d = x.shape
    h32 = h.astype(jnp.int32)
    parts = []
    if SC_ROWS < batch:
        h2 = h32.reshape(d, 1)
        sign2 = sign.reshape(d, 1)
        parts.append(_tc_countsketch(x[: batch - SC_ROWS], h2, sign2))
    if SC_ROWS > 0:
        parts.append(_sc_countsketch(x[batch - SC_ROWS :], h32, sign))
    if len(parts) == 1:
        return parts[0]
    return jnp.concatenate(parts, axis=0)


# speedup vs baseline: 2.7116x; 1.5042x over previous
import dataclasses
import functools

import jax
import jax.numpy as jnp
from jax import lax
from jax.experimental import pallas as pl
from jax.experimental.pallas import tpu as pltpu
from jax.experimental.pallas import tpu_sc as plsc

M = 2048
BATCH_BLOCK = 512
NW = 32
SC_ROWS = 1024


def _tc_body(h_ref, sign_ref, x_ref, out_ref, s_scratch):
    d = h_ref.shape[0]
    step = pl.program_id(0)

    @pl.when(step == 0)
    def _build():
        bins = jax.lax.broadcasted_iota(jnp.int32, (d, M), 1)
        onehot = jnp.where(h_ref[...] == bins, sign_ref[...], 0.0)
        s_scratch[...] = onehot.astype(jnp.bfloat16)

    @pl.when(step > 0)
    def _dot():
        out_ref[...] = jnp.dot(
            x_ref[...].astype(jnp.bfloat16),
            s_scratch[...],
            preferred_element_type=jnp.float32,
        )


def _tc_countsketch(x, h2, sign2):
    batch, d = x.shape
    nb = batch // BATCH_BLOCK
    blk = lambda b: jnp.maximum(b - 1, 0)
    return pl.pallas_call(
        _tc_body,
        grid=(nb + 1,),
        in_specs=[
            pl.BlockSpec((d, 1), lambda b: (0, 0)),
            pl.BlockSpec((d, 1), lambda b: (0, 0)),
            pl.BlockSpec((BATCH_BLOCK, d), lambda b: (blk(b), 0)),
        ],
        out_specs=pl.BlockSpec((BATCH_BLOCK, M), lambda b: (blk(b), 0)),
        out_shape=jax.ShapeDtypeStruct((batch, M), x.dtype),
        scratch_shapes=[pltpu.VMEM((d, M), jnp.bfloat16)],
        compiler_params=pltpu.CompilerParams(
            dimension_semantics=("arbitrary",),
        ),
    )(h2, sign2, x)


ROWS_PER_CHUNK = 8


def _sc_body(x_hbm, h_hbm, sign_hbm, out_hbm, h_v, sign_v, xa_v, xb_v, acc_v,
             sem_a, sem_b):
    d = h_hbm.shape[0]
    wid = lax.axis_index("s") * 2 + lax.axis_index("c")
    rows_per_w = x_hbm.shape[0] // NW
    n_chunks = rows_per_w // ROWS_PER_CHUNK
    base = wid * rows_per_w

    pltpu.sync_copy(h_hbm, h_v)
    pltpu.sync_copy(sign_hbm, sign_v)

    bufs = (xa_v, xb_v)
    sems = (sem_a, sem_b)

    def start(t):
        return pltpu.async_copy(
            x_hbm.at[pl.ds(base + t * ROWS_PER_CHUNK, ROWS_PER_CHUNK)],
            bufs[t % 2],
            sems[t % 2],
        )

    handles = {0: start(0)}
    for t in range(n_chunks):
        if t + 1 < n_chunks:
            handles[t + 1] = start(t + 1)
        handles[t].wait()
        xbuf = bufs[t % 2]

        @pl.loop(0, M, step=16)
        def _zero(i):
            z = jnp.zeros((16,), jnp.float32)
            for r in range(ROWS_PER_CHUNK):
                acc_v[r, pl.ds(i, 16)] = z

        @pl.loop(0, d, step=16)
        def _grp(g):
            idx = h_v[pl.ds(g, 16)]
            sgn = sign_v[pl.ds(g, 16)]
            for r in range(ROWS_PER_CHUNK):
                v = xbuf[r, pl.ds(g, 16)] * sgn
                row_idx = jnp.full((16,), r, jnp.int32)
                plsc.addupdate_scatter(acc_v, [row_idx, idx], v)

        pltpu.sync_copy(
            acc_v,
            out_hbm.at[pl.ds(base + t * ROWS_PER_CHUNK, ROWS_PER_CHUNK)],
        )


def _sc_countsketch(x, h32, sign):
    batch, d = x.shape
    mesh = plsc.VectorSubcoreMesh(core_axis_name="c", subcore_axis_name="s")
    cp = pltpu.CompilerParams()
    if "needs_layout_passes" in pltpu.CompilerParams.__dataclass_fields__:
        cp = dataclasses.replace(cp, needs_layout_passes=False)
    f = functools.partial(
        pl.kernel,
        out_type=jax.ShapeDtypeStruct((batch, M), jnp.float32),
        mesh=mesh,
        compiler_params=cp,
        scratch_types=[
            pltpu.VMEM((d,), jnp.int32),
            pltpu.VMEM((d,), jnp.float32),
            pltpu.VMEM((ROWS_PER_CHUNK, d), jnp.float32),
            pltpu.VMEM((ROWS_PER_CHUNK, d), jnp.float32),
            pltpu.VMEM((ROWS_PER_CHUNK, M), jnp.float32),
            pltpu.SemaphoreType.DMA,
            pltpu.SemaphoreType.DMA,
        ],
    )(_sc_body)
    return f(x, h32, sign)


def kernel(x, h, sign):
    batch, d = x.shape
    h32 = h.astype(jnp.int32)
    parts = []
    if SC_ROWS < batch:
        h2 = h32.reshape(d, 1)
        sign2 = sign.reshape(d, 1)
        parts.append(_tc_countsketch(x[: batch - SC_ROWS], h2, sign2))
    if SC_ROWS > 0:
        parts.append(_sc_countsketch(x[batch - SC_ROWS :], h32, sign))
    if len(parts) == 1:
        return parts[0]
    return jnp.concatenate(parts, axis=0)


# speedup vs baseline: 4.3703x; 1.6117x over previous
import dataclasses
import functools

import jax
import jax.numpy as jnp
from jax import lax
from jax.experimental import pallas as pl
from jax.experimental.pallas import tpu as pltpu
from jax.experimental.pallas import tpu_sc as plsc

M = 2048
BATCH_BLOCK = 512
NW = 32
SC_ROWS = 0


S_BUILD_BLOCK = 512


def _s_build_body(h_ref, sign_ref, s_ref):
    kb = h_ref.shape[0]
    bins = jax.lax.broadcasted_iota(jnp.int32, (kb, M), 1)
    onehot = jnp.where(h_ref[...] == bins, sign_ref[...], 0.0)
    s_ref[...] = onehot.astype(jnp.bfloat16)


def _build_s(h2, sign2):
    d = h2.shape[0]
    nk = d // S_BUILD_BLOCK
    return pl.pallas_call(
        _s_build_body,
        grid=(nk,),
        in_specs=[
            pl.BlockSpec((S_BUILD_BLOCK, 1), lambda k: (k, 0)),
            pl.BlockSpec((S_BUILD_BLOCK, 1), lambda k: (k, 0)),
        ],
        out_specs=pl.BlockSpec((S_BUILD_BLOCK, M), lambda k: (k, 0)),
        out_shape=jax.ShapeDtypeStruct((d, M), jnp.bfloat16),
    )(h2, sign2)


def _tc_body(s_ref, x_ref, out_ref):
    out_ref[...] = jnp.dot(
        x_ref[...].astype(jnp.bfloat16),
        s_ref[...],
        preferred_element_type=jnp.float32,
    )


def _tc_countsketch(x, s):
    batch, d = x.shape
    nb = batch // BATCH_BLOCK
    return pl.pallas_call(
        _tc_body,
        grid=(nb,),
        in_specs=[
            pl.BlockSpec((d, M), lambda b: (0, 0)),
            pl.BlockSpec((BATCH_BLOCK, d), lambda b: (b, 0)),
        ],
        out_specs=pl.BlockSpec((BATCH_BLOCK, M), lambda b: (b, 0)),
        out_shape=jax.ShapeDtypeStruct((batch, M), x.dtype),
        compiler_params=pltpu.CompilerParams(
            dimension_semantics=("arbitrary",),
        ),
    )(s, x)


ROWS_PER_CHUNK = 8


def _sc_body(x_hbm, h_hbm, sign_hbm, out_hbm, h_v, sign_v, xa_v, xb_v, acc_v,
             sem_a, sem_b):
    d = h_hbm.shape[0]
    wid = lax.axis_index("s") * 2 + lax.axis_index("c")
    rows_per_w = x_hbm.shape[0] // NW
    n_chunks = rows_per_w // ROWS_PER_CHUNK
    base = wid * rows_per_w

    pltpu.sync_copy(h_hbm, h_v)
    pltpu.sync_copy(sign_hbm, sign_v)

    bufs = (xa_v, xb_v)
    sems = (sem_a, sem_b)

    def start(t):
        return pltpu.async_copy(
            x_hbm.at[pl.ds(base + t * ROWS_PER_CHUNK, ROWS_PER_CHUNK)],
            bufs[t % 2],
            sems[t % 2],
        )

    handles = {0: start(0)}
    for t in range(n_chunks):
        if t + 1 < n_chunks:
            handles[t + 1] = start(t + 1)
        handles[t].wait()
        xbuf = bufs[t % 2]

        @pl.loop(0, M, step=16)
        def _zero(i):
            z = jnp.zeros((16,), jnp.float32)
            for r in range(ROWS_PER_CHUNK):
                acc_v[r, pl.ds(i, 16)] = z

        @pl.loop(0, d, step=16)
        def _grp(g):
            idx = h_v[pl.ds(g, 16)]
            sgn = sign_v[pl.ds(g, 16)]
            for r in range(ROWS_PER_CHUNK):
                v = xbuf[r, pl.ds(g, 16)] * sgn
                row_idx = jnp.full((16,), r, jnp.int32)
                plsc.addupdate_scatter(acc_v, [row_idx, idx], v)

        pltpu.sync_copy(
            acc_v,
            out_hbm.at[pl.ds(base + t * ROWS_PER_CHUNK, ROWS_PER_CHUNK)],
        )


def _sc_countsketch(x, h32, sign):
    batch, d = x.shape
    mesh = plsc.VectorSubcoreMesh(core_axis_name="c", subcore_axis_name="s")
    cp = pltpu.CompilerParams()
    if "needs_layout_passes" in pltpu.CompilerParams.__dataclass_fields__:
        cp = dataclasses.replace(cp, needs_layout_passes=False)
    f = functools.partial(
        pl.kernel,
        out_type=jax.ShapeDtypeStruct((batch, M), jnp.float32),
        mesh=mesh,
        compiler_params=cp,
        scratch_types=[
            pltpu.VMEM((d,), jnp.int32),
            pltpu.VMEM((d,), jnp.float32),
            pltpu.VMEM((ROWS_PER_CHUNK, d), jnp.float32),
            pltpu.VMEM((ROWS_PER_CHUNK, d), jnp.float32),
            pltpu.VMEM((ROWS_PER_CHUNK, M), jnp.float32),
            pltpu.SemaphoreType.DMA,
            pltpu.SemaphoreType.DMA,
        ],
    )(_sc_body)
    return f(x, h32, sign)


def kernel(x, h, sign):
    batch, d = x.shape
    h32 = h.astype(jnp.int32)
    parts = []
    if SC_ROWS < batch:
        h2 = h32.reshape(d, 1)
        sign2 = sign.reshape(d, 1)
        s = _build_s(h2, sign2)
        parts.append(_tc_countsketch(x[: batch - SC_ROWS], s))
    if SC_ROWS > 0:
        parts.append(_sc_countsketch(x[batch - SC_ROWS :], h32, sign))
    if len(parts) == 1:
        return parts[0]
    return jnp.concatenate(parts, axis=0)


# speedup vs baseline: 4.9602x; 1.1350x over previous
import dataclasses
import functools

import jax
import jax.numpy as jnp
from jax import lax
from jax.experimental import pallas as pl
from jax.experimental.pallas import tpu as pltpu
from jax.experimental.pallas import tpu_sc as plsc

M = 2048
BATCH_BLOCK = 256
NW = 32
SC_ROWS = 0


def _tc_body(h_ref, sign_ref, x_ref, out_ref, s_scratch):
    d = h_ref.shape[0]
    step = pl.program_id(0)

    @pl.when(step == 0)
    def _build():
        bins = jax.lax.broadcasted_iota(jnp.int32, (d, M), 1)
        onehot = jnp.where(h_ref[...] == bins, sign_ref[...], 0.0)
        s_scratch[...] = onehot.astype(jnp.bfloat16)

    @pl.when(step > 0)
    def _dot():
        out_ref[...] = jnp.dot(
            x_ref[...].astype(jnp.bfloat16),
            s_scratch[...],
            preferred_element_type=jnp.float32,
        )


def _tc_countsketch(x, h2, sign2):
    batch, d = x.shape
    nb = batch // BATCH_BLOCK
    blk = lambda b: jnp.maximum(b - 1, 0)
    return pl.pallas_call(
        _tc_body,
        grid=(nb + 1,),
        in_specs=[
            pl.BlockSpec((d, 1), lambda b: (0, 0)),
            pl.BlockSpec((d, 1), lambda b: (0, 0)),
            pl.BlockSpec((BATCH_BLOCK, d), lambda b: (blk(b), 0)),
        ],
        out_specs=pl.BlockSpec((BATCH_BLOCK, M), lambda b: (blk(b), 0)),
        out_shape=jax.ShapeDtypeStruct((batch, M), x.dtype),
        scratch_shapes=[pltpu.VMEM((d, M), jnp.bfloat16)],
        compiler_params=pltpu.CompilerParams(
            dimension_semantics=("arbitrary",),
        ),
    )(h2, sign2, x)


ROWS_PER_CHUNK = 8


def _sc_body(x_hbm, h_hbm, sign_hbm, out_hbm, h_v, sign_v, xa_v, xb_v, acc_v,
             sem_a, sem_b):
    d = h_hbm.shape[0]
    wid = lax.axis_index("s") * 2 + lax.axis_index("c")
    rows_per_w = x_hbm.shape[0] // NW
    n_chunks = rows_per_w // ROWS_PER_CHUNK
    base = wid * rows_per_w

    pltpu.sync_copy(h_hbm, h_v)
    pltpu.sync_copy(sign_hbm, sign_v)

    bufs = (xa_v, xb_v)
    sems = (sem_a, sem_b)

    def start(t):
        return pltpu.async_copy(
            x_hbm.at[pl.ds(base + t * ROWS_PER_CHUNK, ROWS_PER_CHUNK)],
            bufs[t % 2],
            sems[t % 2],
        )

    handles = {0: start(0)}
    for t in range(n_chunks):
        if t + 1 < n_chunks:
            handles[t + 1] = start(t + 1)
        handles[t].wait()
        xbuf = bufs[t % 2]

        @pl.loop(0, M, step=16)
        def _zero(i):
            z = jnp.zeros((16,), jnp.float32)
            for r in range(ROWS_PER_CHUNK):
                acc_v[r, pl.ds(i, 16)] = z

        @pl.loop(0, d, step=16)
        def _grp(g):
            idx = h_v[pl.ds(g, 16)]
            sgn = sign_v[pl.ds(g, 16)]
            for r in range(ROWS_PER_CHUNK):
                v = xbuf[r, pl.ds(g, 16)] * sgn
                row_idx = jnp.full((16,), r, jnp.int32)
                plsc.addupdate_scatter(acc_v, [row_idx, idx], v)

        pltpu.sync_copy(
            acc_v,
            out_hbm.at[pl.ds(base + t * ROWS_PER_CHUNK, ROWS_PER_CHUNK)],
        )


def _sc_countsketch(x, h32, sign):
    batch, d = x.shape
    mesh = plsc.VectorSubcoreMesh(core_axis_name="c", subcore_axis_name="s")
    cp = pltpu.CompilerParams()
    if "needs_layout_passes" in pltpu.CompilerParams.__dataclass_fields__:
        cp = dataclasses.replace(cp, needs_layout_passes=False)
    f = functools.partial(
        pl.kernel,
        out_type=jax.ShapeDtypeStruct((batch, M), jnp.float32),
        mesh=mesh,
        compiler_params=cp,
        scratch_types=[
            pltpu.VMEM((d,), jnp.int32),
            pltpu.VMEM((d,), jnp.float32),
            pltpu.VMEM((ROWS_PER_CHUNK, d), jnp.float32),
            pltpu.VMEM((ROWS_PER_CHUNK, d), jnp.float32),
            pltpu.VMEM((ROWS_PER_CHUNK, M), jnp.float32),
            pltpu.SemaphoreType.DMA,
            pltpu.SemaphoreType.DMA,
        ],
    )(_sc_body)
    return f(x, h32, sign)


def kernel(x, h, sign):
    batch, d = x.shape
    h32 = h.astype(jnp.int32)
    parts = []
    if SC_ROWS < batch:
        h2 = h32.reshape(d, 1)
        sign2 = sign.reshape(d, 1)
        parts.append(_tc_countsketch(x[: batch - SC_ROWS], h2, sign2))
    if SC_ROWS > 0:
        parts.append(_sc_countsketch(x[batch - SC_ROWS :], h32, sign))
    if len(parts) == 1:
        return parts[0]
    return jnp.concatenate(parts, axis=0)
